# baseline (speedup 1.0000x reference)
import functools

import jax
import jax.numpy as jnp
from jax import lax
from jax.experimental import pallas as pl
from jax.experimental.pallas import tpu as pltpu
from jax.experimental.pallas import tpu_sc as plsc

_NC = 2
_NS = 16
_NW = _NC * _NS


def _sc_gather(xf, deep_table, wide_table, lrw2, nsamp):
    BF = xf.shape[0]
    D = deep_table.shape[1]
    F = lrw2.shape[0]
    per_w = BF // _NW
    CH = 1664
    assert per_w % CH == 0 and CH % F == 0
    nch = per_w // CH
    samp_ch = CH // F

    mesh = plsc.VectorSubcoreMesh(
        core_axis_name="c", subcore_axis_name="s", num_cores=_NC, num_subcores=_NS
    )

    @functools.partial(
        pl.kernel,
        out_type=(
            jax.ShapeDtypeStruct((BF, D), jnp.float32),
            jax.ShapeDtypeStruct((nsamp, D), jnp.float32),
        ),
        mesh=mesh,
        compiler_params=pltpu.CompilerParams(use_tc_tiling_on_sc=False),
        scratch_types=[
            pltpu.VMEM((CH,), jnp.int32),
            pltpu.VMEM((CH,), jnp.int32),
            pltpu.VMEM((CH, D), jnp.float32),
            pltpu.VMEM((CH, D), jnp.float32),
            pltpu.VMEM((CH, D), jnp.float32),
            pltpu.VMEM((CH, D), jnp.float32),
            pltpu.VMEM((samp_ch, D), jnp.float32),
            pltpu.VMEM((F, D), jnp.float32),
            pltpu.SemaphoreType.DMA,
            pltpu.SemaphoreType.DMA,
            pltpu.SemaphoreType.DMA,
            pltpu.SemaphoreType.DMA,
            pltpu.SemaphoreType.DMA,
            pltpu.SemaphoreType.DMA,
        ],
    )
    def gather_k(idx_hbm, deep_hbm, wide_hbm, lrw_hbm, dout, wout,
                 idx0, idx1, dr0, dr1, wr0, wr1, wpart_v, lrw_v,
                 semd0, semd1, semw0, semw1, semb0, semb1):
        wid = lax.axis_index("s") * _NC + lax.axis_index("c")
        idx_b = (idx0, idx1)
        dr_b = (dr0, dr1)
        wr_b = (wr0, wr1)
        semd_b = (semd0, semd1)
        semw_b = (semw0, semw1)
        semb_b = (semb0, semb1)

        pltpu.sync_copy(lrw_hbm, lrw_v)

        def stage(c, b):
            base = wid * per_w + c * CH
            pltpu.sync_copy(idx_hbm.at[pl.ds(base, CH)], idx_b[b])
            cp_d = pltpu.async_copy(deep_hbm.at[idx_b[b]], dr_b[b], semd_b[b])
            cp_w = pltpu.async_copy(wide_hbm.at[idx_b[b]], wr_b[b], semw_b[b])
            return cp_d, cp_w

        inflight = {}
        wb = {}
        inflight[0] = stage(0, 0)
        if nch > 1:
            inflight[1] = stage(1, 1)
        for c in range(nch):
            b = c % 2
            base = wid * per_w + c * CH
            sbase = wid * (per_w // F) + c * samp_ch
            cp_d, cp_w = inflight.pop(c)
            cp_d.wait()
            wb[c] = pltpu.async_copy(dr_b[b], dout.at[pl.ds(base, CH)], semb_b[b])
            cp_w.wait()

            wrows = wr_b[b]

            def body(s, _):
                r0 = s * F
                acc = wrows[r0] * lrw_v[0]
                for f in range(1, F):
                    acc = acc + wrows[r0 + f] * lrw_v[f]
                wpart_v[s] = acc
                return 0

            lax.fori_loop(0, samp_ch, body, 0)
            pltpu.sync_copy(wpart_v, wout.at[pl.ds(sbase, samp_ch)])
            if c + 2 < nch:
                wb.pop(c).wait()
                inflight[c + 2] = stage(c + 2, b)
        for c in sorted(wb):
            wb.pop(c).wait()

    return gather_k(xf, deep_table, wide_table, lrw2)


_BN = 16384


def _transpose_rows(x, w, m):
    z = lax.dot_general(x, w, (((0,), (0,)), ((), ())),
                        precision=lax.Precision.HIGHEST,
                        preferred_element_type=jnp.float32)
    z3 = jnp.reshape(z, (_BN // 8, 8, 128))
    return jnp.sum(z3 * m[None, :, :], axis=1)


def _detile_body(d_ref, w_ref, wsel_ref, msel_ref, od_ref, ow_ref):
    od_ref[...] = _transpose_rows(d_ref[...], wsel_ref[...], msel_ref[...])
    ow_ref[...] = _transpose_rows(w_ref[...], wsel_ref[...], msel_ref[...])


def _tc_detile(dT, wT):
    D, V = dT.shape
    grid = V // _BN
    j = jnp.arange(128)
    wsel = (j[None, :] % D == jnp.arange(D)[:, None]).astype(jnp.float32)
    msel = (j[None, :] // D == jnp.arange(8)[:, None]).astype(jnp.float32)
    return pl.pallas_call(
        _detile_body,
        grid=(grid,),
        in_specs=[
            pl.BlockSpec((D, _BN), lambda i: (0, i)),
            pl.BlockSpec((D, _BN), lambda i: (0, i)),
            pl.BlockSpec((D, 128), lambda i: (0, 0)),
            pl.BlockSpec((8, 128), lambda i: (0, 0)),
        ],
        out_specs=[
            pl.BlockSpec((_BN // 8, 128), lambda i: (i, 0)),
            pl.BlockSpec((_BN // 8, 128), lambda i: (i, 0)),
        ],
        out_shape=[
            jax.ShapeDtypeStruct((V // 8, 128), jnp.float32),
            jax.ShapeDtypeStruct((V // 8, 128), jnp.float32),
        ],
    )(dT, wT, wsel, msel)


def _tc_body(deep_ref, wpart_ref, s_ref, w1_ref, b1_ref, w2_ref, b2_ref,
             w3_ref, b3_ref, w4_ref, bout_ref, o_ref):
    xb = deep_ref[...]
    sum_v = jnp.dot(xb, s_ref[...], preferred_element_type=jnp.float32)
    term1 = jnp.sum(sum_v * sum_v, axis=1, keepdims=True)
    term2 = jnp.sum(xb * xb, axis=1, keepdims=True)
    fm = 0.5 * (term1 - term2)
    lr = jnp.sum(wpart_ref[...], axis=1, keepdims=True)
    h = jnp.maximum(
        jnp.dot(xb, w1_ref[...], preferred_element_type=jnp.float32) + b1_ref[...], 0.0)
    h = jnp.maximum(
        jnp.dot(h, w2_ref[...], preferred_element_type=jnp.float32) + b2_ref[...], 0.0)
    h = jnp.maximum(
        jnp.dot(h, w3_ref[...], preferred_element_type=jnp.float32) + b3_ref[...], 0.0)
    dnn = jnp.dot(h, w4_ref[...], preferred_element_type=jnp.float32)
    o_ref[...] = jax.nn.sigmoid(fm + lr + dnn + bout_ref[...])


def _tc_mlp(nn_map, wpart, S, W1, b1, W2, b2, W3, b3, W4, bias_out):
    B, K = nn_map.shape
    D = wpart.shape[1]
    bm = 2048
    grid = B // bm
    d1 = W1.shape[1]
    d2 = W2.shape[1]
    d3 = W3.shape[1]

    full = lambda shape: pl.BlockSpec(shape, lambda i: (0, 0))
    return pl.pallas_call(
        _tc_body,
        grid=(grid,),
        in_specs=[
            pl.BlockSpec((bm, K), lambda i: (i, 0)),
            pl.BlockSpec((bm, D), lambda i: (i, 0)),
            full(S.shape),
            full(W1.shape),
            full((1, d1)),
            full(W2.shape),
            full((1, d2)),
            full(W3.shape),
            full((1, d3)),
            full(W4.shape),
            full((1, 1)),
        ],
        out_specs=pl.BlockSpec((bm, 1), lambda i: (i, 0)),
        out_shape=jax.ShapeDtypeStruct((B, 1), jnp.float32),
    )(nn_map, wpart, S, W1, b1.reshape(1, d1), W2, b2.reshape(1, d2),
      W3, b3.reshape(1, d3), W4, bias_out)


def kernel(x, deep_table, wide_table, lr_W, lr_b, W1, b1, W2, b2, W3, b3, W4, b4):
    B, F = x.shape
    D = deep_table.shape[1]
    xf = x.reshape(-1).astype(jnp.int32)
    lrw2 = lr_W.reshape(F, D)
    V = deep_table.shape[0]
    VP = ((V + _BN - 1) // _BN) * _BN
    dTp = jnp.pad(deep_table, ((0, VP - V), (0, 0))).T
    wTp = jnp.pad(wide_table, ((0, VP - V), (0, 0))).T
    dl, wl = _tc_detile(dTp, wTp)
    deep_rows, wpart = _sc_gather(
        xf, dl.reshape(VP, D), wl.reshape(VP, D), lrw2, B)
    nn_map = deep_rows.reshape(B, F * D)
    S = jnp.tile(jnp.eye(D, dtype=jnp.float32), (F, 1))
    bias_out = (b4 + lr_b).reshape(1, 1)
    return _tc_mlp(nn_map, wpart, S, W1, b1, W2, b2, W3, b3, W4, bias_out)


# speedup vs baseline: 1.4705x; 1.4705x over previous
import functools

import jax
import jax.numpy as jnp
from jax import lax
from jax.experimental import pallas as pl
from jax.experimental.pallas import tpu as pltpu
from jax.experimental.pallas import tpu_sc as plsc

_NC = 2
_NS = 16
_NW = _NC * _NS


def _sc_gather(xf, deep_table, wide_table, lrw2, nsamp):
    BF = xf.shape[0]
    D = deep_table.shape[1]
    F = lrw2.shape[0]
    per_w = BF // _NW
    CH = 1664
    assert per_w % CH == 0 and CH % F == 0
    nch = per_w // CH
    samp_ch = CH // F

    mesh = plsc.VectorSubcoreMesh(
        core_axis_name="c", subcore_axis_name="s", num_cores=_NC, num_subcores=_NS
    )

    @functools.partial(
        pl.kernel,
        out_type=(
            jax.ShapeDtypeStruct((BF, D), jnp.float32),
            jax.ShapeDtypeStruct((nsamp, D), jnp.float32),
        ),
        mesh=mesh,
        compiler_params=pltpu.CompilerParams(use_tc_tiling_on_sc=False),
        scratch_types=[
            pltpu.VMEM((CH,), jnp.int32),
            pltpu.VMEM((CH,), jnp.int32),
            pltpu.VMEM((CH, D), jnp.float32),
            pltpu.VMEM((CH, D), jnp.float32),
            pltpu.VMEM((CH, D), jnp.float32),
            pltpu.VMEM((CH, D), jnp.float32),
            pltpu.VMEM((samp_ch, D), jnp.float32),
            pltpu.VMEM((F, D), jnp.float32),
            pltpu.SemaphoreType.DMA,
            pltpu.SemaphoreType.DMA,
            pltpu.SemaphoreType.DMA,
            pltpu.SemaphoreType.DMA,
            pltpu.SemaphoreType.DMA,
            pltpu.SemaphoreType.DMA,
        ],
    )
    def gather_k(idx_hbm, deep_hbm, wide_hbm, lrw_hbm, dout, wout,
                 idx0, idx1, dr0, dr1, wr0, wr1, wpart_v, lrw_v,
                 semd0, semd1, semw0, semw1, semb0, semb1):
        wid = lax.axis_index("s") * _NC + lax.axis_index("c")
        idx_b = (idx0, idx1)
        dr_b = (dr0, dr1)
        wr_b = (wr0, wr1)
        semd_b = (semd0, semd1)
        semw_b = (semw0, semw1)
        semb_b = (semb0, semb1)

        pltpu.sync_copy(lrw_hbm, lrw_v)

        def stage(c, b):
            base = wid * per_w + c * CH
            pltpu.sync_copy(idx_hbm.at[pl.ds(base, CH)], idx_b[b])
            cp_d = pltpu.async_copy(deep_hbm.at[idx_b[b]], dr_b[b], semd_b[b])
            cp_w = pltpu.async_copy(wide_hbm.at[idx_b[b]], wr_b[b], semw_b[b])
            return cp_d, cp_w

        inflight = {}
        wb = {}
        inflight[0] = stage(0, 0)
        if nch > 1:
            inflight[1] = stage(1, 1)
        for c in range(nch):
            b = c % 2
            base = wid * per_w + c * CH
            sbase = wid * (per_w // F) + c * samp_ch
            cp_d, cp_w = inflight.pop(c)
            cp_d.wait()
            wb[c] = pltpu.async_copy(dr_b[b], dout.at[pl.ds(base, CH)], semb_b[b])
            cp_w.wait()

            wrows = wr_b[b]

            def body(s, _):
                r0 = s * F
                acc = wrows[r0] * lrw_v[0]
                for f in range(1, F):
                    acc = acc + wrows[r0 + f] * lrw_v[f]
                wpart_v[s] = acc
                return 0

            lax.fori_loop(0, samp_ch, body, 0)
            pltpu.sync_copy(wpart_v, wout.at[pl.ds(sbase, samp_ch)])
            if c + 2 < nch:
                wb.pop(c).wait()
                inflight[c + 2] = stage(c + 2, b)
        for c in sorted(wb):
            wb.pop(c).wait()

    return gather_k(xf, deep_table, wide_table, lrw2)


def _tc_body(deep_ref, wpart_ref, s_ref, w1_ref, b1_ref, w2_ref, b2_ref,
             w3_ref, b3_ref, w4_ref, bout_ref, o_ref):
    xb = deep_ref[...]
    sum_v = jnp.dot(xb, s_ref[...], preferred_element_type=jnp.float32)
    term1 = jnp.sum(sum_v * sum_v, axis=1, keepdims=True)
    term2 = jnp.sum(xb * xb, axis=1, keepdims=True)
    fm = 0.5 * (term1 - term2)
    lr = jnp.sum(wpart_ref[...], axis=1, keepdims=True)
    h = jnp.maximum(
        jnp.dot(xb, w1_ref[...], preferred_element_type=jnp.float32) + b1_ref[...], 0.0)
    h = jnp.maximum(
        jnp.dot(h, w2_ref[...], preferred_element_type=jnp.float32) + b2_ref[...], 0.0)
    h = jnp.maximum(
        jnp.dot(h, w3_ref[...], preferred_element_type=jnp.float32) + b3_ref[...], 0.0)
    dnn = jnp.dot(h, w4_ref[...], preferred_element_type=jnp.float32)
    o_ref[...] = jax.nn.sigmoid(fm + lr + dnn + bout_ref[...])


def _tc_mlp(nn_map, wpart, S, W1, b1, W2, b2, W3, b3, W4, bias_out):
    B, K = nn_map.shape
    D = wpart.shape[1]
    bm = 2048
    grid = B // bm
    d1 = W1.shape[1]
    d2 = W2.shape[1]
    d3 = W3.shape[1]

    full = lambda shape: pl.BlockSpec(shape, lambda i: (0, 0))
    return pl.pallas_call(
        _tc_body,
        grid=(grid,),
        in_specs=[
            pl.BlockSpec((bm, K), lambda i: (i, 0)),
            pl.BlockSpec((bm, D), lambda i: (i, 0)),
            full(S.shape),
            full(W1.shape),
            full((1, d1)),
            full(W2.shape),
            full((1, d2)),
            full(W3.shape),
            full((1, d3)),
            full(W4.shape),
            full((1, 1)),
        ],
        out_specs=pl.BlockSpec((bm, 1), lambda i: (i, 0)),
        out_shape=jax.ShapeDtypeStruct((B, 1), jnp.float32),
    )(nn_map, wpart, S, W1, b1.reshape(1, d1), W2, b2.reshape(1, d2),
      W3, b3.reshape(1, d3), W4, bias_out)


def kernel(x, deep_table, wide_table, lr_W, lr_b, W1, b1, W2, b2, W3, b3, W4, b4):
    B, F = x.shape
    D = deep_table.shape[1]
    xf = x.reshape(-1).astype(jnp.int32)
    lrw2 = lr_W.reshape(F, D)
    deep_rows, wpart = _sc_gather(xf, deep_table, wide_table, lrw2, B)
    nn_map = deep_rows.reshape(B, F * D)
    S = jnp.tile(jnp.eye(D, dtype=jnp.float32), (F, 1))
    bias_out = (b4 + lr_b).reshape(1, 1)
    return _tc_mlp(nn_map, wpart, S, W1, b1, W2, b2, W3, b3, W4, bias_out)
